# Initial kernel scaffold; baseline (speedup 1.0000x reference)
import functools

import jax
import jax.numpy as jnp
from jax import lax
from jax.experimental import pallas as pl
from jax.experimental.pallas import tpu as pltpu
from jax.experimental.pallas import tpu_sc as plsc

N = 10000
E = 320000
D = 128

NC = 2
NS = 16
NW = NC * NS

CH = 80
E_PER_W = E // NW
N_CHUNKS_E = E_PER_W // CH
N_CHUNKS_N = N // CH


def _linear_body(x_ref, wt_ref, b_ref, out_ref):
    out_ref[...] = (
        jnp.dot(x_ref[...], wt_ref[...], preferred_element_type=jnp.float32)
        + b_ref[...]
    )


def _linear(x, wt, b2d):
    grid = 10
    blk = N // grid
    return pl.pallas_call(
        _linear_body,
        grid=(grid,),
        in_specs=[
            pl.BlockSpec((blk, D), lambda i: (i, 0)),
            pl.BlockSpec((D, D), lambda i: (0, 0)),
            pl.BlockSpec((1, D), lambda i: (0, 0)),
        ],
        out_specs=pl.BlockSpec((blk, D), lambda i: (i, 0)),
        out_shape=jax.ShapeDtypeStruct((N, D), jnp.float32),
    )(x, wt, b2d)


def _sc_body(h_hbm, col_hbm, row_hbm, ew_hbm, prev_hbm,
             partial_hbm, prevout_hbm,
             colv, rowv, wv, rows, acc, sem):
    cid = lax.axis_index("c")
    sid = lax.axis_index("s")
    wid = cid * NS + sid

    zero16 = jnp.zeros((16,), jnp.float32)

    def zrow(i, carry):
        for j in range(8):
            rows[i, pl.ds(j * 16, 16)] = zero16
        return carry

    lax.fori_loop(0, CH, zrow, 0)

    for r in range(8):
        c = sid + NS * r

        @pl.when(c < N_CHUNKS_N)
        def _():
            pltpu.sync_copy(rows, acc.at[pl.ds(c * CH, CH)])

    plsc.subcore_barrier()

    base = wid * E_PER_W

    def chunk_body(k, carry):
        off = base + k * CH
        pltpu.sync_copy(col_hbm.at[pl.ds(off, CH)], colv)
        pltpu.sync_copy(row_hbm.at[pl.ds(off, CH)], rowv)
        pltpu.sync_copy(ew_hbm.at[pl.ds(off, CH)], wv)
        pltpu.async_copy(h_hbm.at[colv], rows, sem).wait()

        def mul_body(i, mc):
            ws = jnp.full((16,), wv[i], jnp.float32)
            for j in range(8):
                sl = pl.ds(j * 16, 16)
                rows[i, sl] = rows[i, sl] * ws
            return mc

        lax.fori_loop(0, CH, mul_body, 0)
        pltpu.sync_copy(rows, acc.at[rowv], add=True)
        return carry

    lax.fori_loop(0, N_CHUNKS_E, chunk_body, 0)

    for r in range(4):
        c = wid + NW * r

        @pl.when(c < N_CHUNKS_N)
        def _():
            pltpu.sync_copy(prev_hbm.at[pl.ds(c * CH, CH)], colv)
            pltpu.async_copy(h_hbm.at[colv], rows, sem).wait()
            pltpu.sync_copy(rows, prevout_hbm.at[pl.ds(c * CH, CH)])

    plsc.subcore_barrier()

    for r in range(8):
        c = sid + NS * r

        @pl.when(c < N_CHUNKS_N)
        def _():
            pltpu.sync_copy(acc.at[pl.ds(c * CH, CH)], rows)
            pltpu.sync_copy(rows, partial_hbm.at[cid, pl.ds(c * CH, CH)])


_sc_aggregate = functools.partial(
    pl.kernel,
    out_type=[
        jax.ShapeDtypeStruct((NC, N, D), jnp.float32),
        jax.ShapeDtypeStruct((N, D), jnp.float32),
    ],
    mesh=plsc.VectorSubcoreMesh(
        core_axis_name="c", subcore_axis_name="s", num_cores=NC, num_subcores=NS
    ),
    scratch_types=[
        pltpu.VMEM((CH,), jnp.int32),
        pltpu.VMEM((CH,), jnp.int32),
        pltpu.VMEM((CH,), jnp.float32),
        pltpu.VMEM((CH, D), jnp.float32),
        pltpu.VMEM_SHARED((N, D), jnp.float32),
        pltpu.SemaphoreType.DMA,
    ],
)(_sc_body)


def _combine_body(prev_ref, p0_ref, p1_ref, out_ref):
    out_ref[:, :D] = prev_ref[...]
    out_ref[:, D:] = p0_ref[...] + p1_ref[...]


def _combine(prev, p0, p1):
    grid = 10
    blk = N // grid
    return pl.pallas_call(
        _combine_body,
        grid=(grid,),
        in_specs=[
            pl.BlockSpec((blk, D), lambda i: (i, 0)),
            pl.BlockSpec((blk, D), lambda i: (i, 0)),
            pl.BlockSpec((blk, D), lambda i: (i, 0)),
        ],
        out_specs=pl.BlockSpec((blk, 2 * D), lambda i: (i, 0)),
        out_shape=jax.ShapeDtypeStruct((N, 2 * D), jnp.float32),
    )(prev, p0, p1)


def kernel(x, edge_index, edge_weight, previous_index, W, b):
    h = _linear(x, W.T, b.reshape(1, D))
    row = edge_index[0]
    col = edge_index[1]
    partial, prevout = _sc_aggregate(h, col, row, edge_weight, previous_index)
    return _combine(prevout, partial[0], partial[1])


# speedup vs baseline: 4.2873x; 4.2873x over previous
import functools

import jax
import jax.numpy as jnp
from jax import lax
from jax.experimental import pallas as pl
from jax.experimental.pallas import tpu as pltpu
from jax.experimental.pallas import tpu_sc as plsc

N = 10000
E = 320000
D = 128

NC = 2
NS = 16
NW = NC * NS

CH = 80
E_PER_W = E // NW
N_CHUNKS_E = E_PER_W // CH
N_CHUNKS_N = N // CH


def _linear_body(x_ref, wt_ref, b_ref, out_ref):
    out_ref[...] = (
        jnp.dot(x_ref[...], wt_ref[...], preferred_element_type=jnp.float32)
        + b_ref[...]
    )


def _linear(x, wt, b2d):
    grid = 10
    blk = N // grid
    return pl.pallas_call(
        _linear_body,
        grid=(grid,),
        in_specs=[
            pl.BlockSpec((blk, D), lambda i: (i, 0)),
            pl.BlockSpec((D, D), lambda i: (0, 0)),
            pl.BlockSpec((1, D), lambda i: (0, 0)),
        ],
        out_specs=pl.BlockSpec((blk, D), lambda i: (i, 0)),
        out_shape=jax.ShapeDtypeStruct((N, D), jnp.float32),
    )(x, wt, b2d)


def _sc_body(h_hbm, col_hbm, row_hbm, ew_hbm, prev_hbm,
             partial_hbm, prevout_hbm,
             colv, rowv, wv, rows, acc, sem):
    cid = lax.axis_index("c")
    sid = lax.axis_index("s")
    wid = cid * NS + sid

    zero16 = jnp.zeros((16,), jnp.float32)

    def zrow(i, carry):
        for j in range(8):
            rows[i, pl.ds(j * 16, 16)] = zero16
        return carry

    lax.fori_loop(0, CH, zrow, 0)

    for r in range(8):
        c = sid + NS * r

        @pl.when(c < N_CHUNKS_N)
        def _():
            pltpu.sync_copy(rows, acc.at[pl.ds(c * CH, CH)])

    plsc.subcore_barrier()

    base = wid * E_PER_W

    def chunk_body(k, carry):
        off = base + k * CH
        pltpu.sync_copy(col_hbm.at[pl.ds(off, CH)], colv)
        pltpu.sync_copy(row_hbm.at[pl.ds(off, CH)], rowv)
        pltpu.sync_copy(ew_hbm.at[pl.ds(off, CH)], wv)
        pltpu.async_copy(h_hbm.at[colv], rows, sem).wait()

        def mul_group(g, mc):
            w16 = wv[pl.ds(g * 16, 16)]
            base_r = g * 16
            for e in range(16):
                ws = jnp.full((16,), w16[e], jnp.float32)
                r_i = base_r + e
                for j in range(8):
                    sl = pl.ds(j * 16, 16)
                    rows[r_i, sl] = rows[r_i, sl] * ws
            return mc

        lax.fori_loop(0, CH // 16, mul_group, 0)
        pltpu.sync_copy(rows, acc.at[rowv], add=True)
        return carry

    lax.fori_loop(0, N_CHUNKS_E, chunk_body, 0)

    for r in range(4):
        c = wid + NW * r

        @pl.when(c < N_CHUNKS_N)
        def _():
            pltpu.sync_copy(prev_hbm.at[pl.ds(c * CH, CH)], colv)
            pltpu.async_copy(h_hbm.at[colv], rows, sem).wait()
            pltpu.sync_copy(rows, prevout_hbm.at[pl.ds(c * CH, CH)])

    plsc.subcore_barrier()

    for r in range(8):
        c = sid + NS * r

        @pl.when(c < N_CHUNKS_N)
        def _():
            pltpu.sync_copy(acc.at[pl.ds(c * CH, CH)], rows)
            pltpu.sync_copy(rows, partial_hbm.at[cid, pl.ds(c * CH, CH)])


_sc_aggregate = functools.partial(
    pl.kernel,
    out_type=[
        jax.ShapeDtypeStruct((NC, N, D), jnp.float32),
        jax.ShapeDtypeStruct((N, D), jnp.float32),
    ],
    mesh=plsc.VectorSubcoreMesh(
        core_axis_name="c", subcore_axis_name="s", num_cores=NC, num_subcores=NS
    ),
    scratch_types=[
        pltpu.VMEM((CH,), jnp.int32),
        pltpu.VMEM((CH,), jnp.int32),
        pltpu.VMEM((CH,), jnp.float32),
        pltpu.VMEM((CH, D), jnp.float32),
        pltpu.VMEM_SHARED((N, D), jnp.float32),
        pltpu.SemaphoreType.DMA,
    ],
)(_sc_body)


def _combine_body(prev_ref, p0_ref, p1_ref, out_ref):
    out_ref[:, :D] = prev_ref[...]
    out_ref[:, D:] = p0_ref[...] + p1_ref[...]


def _combine(prev, p0, p1):
    grid = 10
    blk = N // grid
    return pl.pallas_call(
        _combine_body,
        grid=(grid,),
        in_specs=[
            pl.BlockSpec((blk, D), lambda i: (i, 0)),
            pl.BlockSpec((blk, D), lambda i: (i, 0)),
            pl.BlockSpec((blk, D), lambda i: (i, 0)),
        ],
        out_specs=pl.BlockSpec((blk, 2 * D), lambda i: (i, 0)),
        out_shape=jax.ShapeDtypeStruct((N, 2 * D), jnp.float32),
    )(prev, p0, p1)


def kernel(x, edge_index, edge_weight, previous_index, W, b):
    h = _linear(x, W.T, b.reshape(1, D))
    row = edge_index[0]
    col = edge_index[1]
    partial, prevout = _sc_aggregate(h, col, row, edge_weight, previous_index)
    return _combine(prevout, partial[0], partial[1])


# speedup vs baseline: 9.7456x; 2.2731x over previous
import functools

import jax
import jax.numpy as jnp
from jax import lax
from jax.experimental import pallas as pl
from jax.experimental.pallas import tpu as pltpu
from jax.experimental.pallas import tpu_sc as plsc

N = 10000
E = 320000
D = 128

NC = 2
NS = 16
NW = NC * NS

CH = 80
E_PER_W = E // NW
N_CHUNKS_E = E_PER_W // CH
N_CHUNKS_N = N // CH


def _linear_body(x_ref, wt_ref, b_ref, out_ref):
    out_ref[...] = (
        jnp.dot(x_ref[...], wt_ref[...], preferred_element_type=jnp.float32)
        + b_ref[...]
    )


def _linear(x, wt, b2d):
    grid = 10
    blk = N // grid
    return pl.pallas_call(
        _linear_body,
        grid=(grid,),
        in_specs=[
            pl.BlockSpec((blk, D), lambda i: (i, 0)),
            pl.BlockSpec((D, D), lambda i: (0, 0)),
            pl.BlockSpec((1, D), lambda i: (0, 0)),
        ],
        out_specs=pl.BlockSpec((blk, D), lambda i: (i, 0)),
        out_shape=jax.ShapeDtypeStruct((N, D), jnp.float32),
    )(x, wt, b2d)


def _sc_body(h_hbm, col_hbm, row_hbm, ew_hbm, prev_hbm,
             partial_hbm, prevout_hbm,
             c0, c1, c2, r0, r1, r2, w0, w1, w2,
             rows0, rows1, acc,
             i0, i1, i2, g0, g1, psem):
    cset = (c0, c1, c2)
    rset = (r0, r1, r2)
    wset = (w0, w1, w2)
    rowsb = (rows0, rows1)
    isem = (i0, i1, i2)
    gsem = (g0, g1)

    cid = lax.axis_index("c")
    sid = lax.axis_index("s")
    wid = cid * NS + sid
    base = wid * E_PER_W

    zero16 = jnp.zeros((16,), jnp.float32)

    def zrow(i, carry):
        for j in range(8):
            rows0[i, pl.ds(j * 16, 16)] = zero16
        return carry

    lax.fori_loop(0, CH, zrow, 0)

    for r in range(8):
        c = sid + NS * r

        @pl.when(c < N_CHUNKS_N)
        def _():
            pltpu.sync_copy(rows0, acc.at[pl.ds(c * CH, CH)])

    plsc.subcore_barrier()

    def idx_issue(k, s):
        off = base + k * CH
        pltpu.async_copy(col_hbm.at[pl.ds(off, CH)], cset[s], isem[s])
        pltpu.async_copy(row_hbm.at[pl.ds(off, CH)], rset[s], isem[s])
        pltpu.async_copy(ew_hbm.at[pl.ds(off, CH)], wset[s], isem[s])

    def idx_wait(k, s):
        off = base + k * CH
        pltpu.make_async_copy(col_hbm.at[pl.ds(off, CH)], cset[s], isem[s]).wait()
        pltpu.make_async_copy(row_hbm.at[pl.ds(off, CH)], rset[s], isem[s]).wait()
        pltpu.make_async_copy(ew_hbm.at[pl.ds(off, CH)], wset[s], isem[s]).wait()

    def mul(buf, wv):
        def grp(g, mc):
            w16 = wv[pl.ds(g * 16, 16)]
            for e in range(16):
                ws = jnp.full((16,), w16[e], jnp.float32)
                r_i = g * 16 + e
                for j in range(8):
                    sl = pl.ds(j * 16, 16)
                    buf[r_i, sl] = buf[r_i, sl] * ws
            return mc

        lax.fori_loop(0, CH // 16, grp, 0)

    def step(k, s, rb, do_next_gather, do_idx):
        if do_next_gather:
            idx_wait(k + 1, (s + 1) % 3)
            pltpu.async_copy(h_hbm.at[cset[(s + 1) % 3]], rowsb[1 - rb],
                             gsem[1 - rb])
        pltpu.make_async_copy(h_hbm.at[cset[s]], rowsb[rb], gsem[rb]).wait()
        mul(rowsb[rb], wset[s])
        pltpu.sync_copy(rowsb[rb], acc.at[rset[s]], add=True)
        if do_idx:
            idx_issue(k + 3, s)

    pltpu.sync_copy(col_hbm.at[pl.ds(base, CH)], c0)
    pltpu.sync_copy(row_hbm.at[pl.ds(base, CH)], r0)
    pltpu.sync_copy(ew_hbm.at[pl.ds(base, CH)], w0)
    pltpu.async_copy(h_hbm.at[c0], rows0, g0)
    idx_issue(1, 1)
    idx_issue(2, 2)

    def six_body(i, carry):
        k = 6 * i
        for off in range(6):
            step(k + off, off % 3, off % 2, True, True)
        return carry

    lax.fori_loop(0, (N_CHUNKS_E - 5) // 6, six_body, 0)

    for k in range((N_CHUNKS_E // 6) * 6, N_CHUNKS_E):
        step(k, k % 3, k % 2, k < N_CHUNKS_E - 1, k + 3 < N_CHUNKS_E)

    for r in range(4):
        c = wid + NW * r

        @pl.when(c < N_CHUNKS_N)
        def _():
            pltpu.sync_copy(prev_hbm.at[pl.ds(c * CH, CH)], c0)
            pltpu.async_copy(h_hbm.at[c0], rows0, psem).wait()
            pltpu.sync_copy(rows0, prevout_hbm.at[pl.ds(c * CH, CH)])

    plsc.subcore_barrier()

    for r in range(8):
        c = sid + NS * r

        @pl.when(c < N_CHUNKS_N)
        def _():
            pltpu.sync_copy(acc.at[pl.ds(c * CH, CH)], rows0)
            pltpu.sync_copy(rows0, partial_hbm.at[cid, pl.ds(c * CH, CH)])


_sc_aggregate = functools.partial(
    pl.kernel,
    out_type=[
        jax.ShapeDtypeStruct((NC, N, D), jnp.float32),
        jax.ShapeDtypeStruct((N, D), jnp.float32),
    ],
    mesh=plsc.VectorSubcoreMesh(
        core_axis_name="c", subcore_axis_name="s", num_cores=NC, num_subcores=NS
    ),
    scratch_types=[
        pltpu.VMEM((CH,), jnp.int32),
        pltpu.VMEM((CH,), jnp.int32),
        pltpu.VMEM((CH,), jnp.int32),
        pltpu.VMEM((CH,), jnp.int32),
        pltpu.VMEM((CH,), jnp.int32),
        pltpu.VMEM((CH,), jnp.int32),
        pltpu.VMEM((CH,), jnp.float32),
        pltpu.VMEM((CH,), jnp.float32),
        pltpu.VMEM((CH,), jnp.float32),
        pltpu.VMEM((CH, D), jnp.float32),
        pltpu.VMEM((CH, D), jnp.float32),
        pltpu.VMEM_SHARED((N, D), jnp.float32),
        pltpu.SemaphoreType.DMA,
        pltpu.SemaphoreType.DMA,
        pltpu.SemaphoreType.DMA,
        pltpu.SemaphoreType.DMA,
        pltpu.SemaphoreType.DMA,
        pltpu.SemaphoreType.DMA,
    ],
)(_sc_body)


def _combine_body(prev_ref, p0_ref, p1_ref, out_ref):
    out_ref[:, :D] = prev_ref[...]
    out_ref[:, D:] = p0_ref[...] + p1_ref[...]


def _combine(prev, p0, p1):
    grid = 10
    blk = N // grid
    return pl.pallas_call(
        _combine_body,
        grid=(grid,),
        in_specs=[
            pl.BlockSpec((blk, D), lambda i: (i, 0)),
            pl.BlockSpec((blk, D), lambda i: (i, 0)),
            pl.BlockSpec((blk, D), lambda i: (i, 0)),
        ],
        out_specs=pl.BlockSpec((blk, 2 * D), lambda i: (i, 0)),
        out_shape=jax.ShapeDtypeStruct((N, 2 * D), jnp.float32),
    )(prev, p0, p1)


def kernel(x, edge_index, edge_weight, previous_index, W, b):
    h = _linear(x, W.T, b.reshape(1, D))
    row = edge_index[0]
    col = edge_index[1]
    partial, prevout = _sc_aggregate(h, col, row, edge_weight, previous_index)
    return _combine(prevout, partial[0], partial[1])


# speedup vs baseline: 10.6801x; 1.0959x over previous
import functools

import jax
import jax.numpy as jnp
from jax import lax
from jax.experimental import pallas as pl
from jax.experimental.pallas import tpu as pltpu
from jax.experimental.pallas import tpu_sc as plsc

N = 10000
E = 320000
D = 128

NC = 2
NS = 16
NW = NC * NS

CH = 80
E_PER_W = E // NW
N_CHUNKS_E = E_PER_W // CH
N_CHUNKS_N = N // CH


def _linear_body(x_ref, wt_ref, b_ref, out_ref):
    out_ref[...] = (
        jnp.dot(x_ref[...], wt_ref[...], preferred_element_type=jnp.float32)
        + b_ref[...]
    )


def _linear(x, wt, b2d):
    grid = 10
    blk = N // grid
    return pl.pallas_call(
        _linear_body,
        grid=(grid,),
        in_specs=[
            pl.BlockSpec((blk, D), lambda i: (i, 0)),
            pl.BlockSpec((D, D), lambda i: (0, 0)),
            pl.BlockSpec((1, D), lambda i: (0, 0)),
        ],
        out_specs=pl.BlockSpec((blk, D), lambda i: (i, 0)),
        out_shape=jax.ShapeDtypeStruct((N, D), jnp.float32),
    )(x, wt, b2d)


def _sc_body(h_hbm, ei_hbm, ew_hbm, prev_hbm,
             partial_hbm, prevout_hbm,
             c0, c1, c2, r0, r1, r2, w0, w1, w2,
             rows0, rows1, acc,
             i0, i1, i2, g0, g1, psem):
    cset = (c0, c1, c2)
    rset = (r0, r1, r2)
    wset = (w0, w1, w2)
    rowsb = (rows0, rows1)
    isem = (i0, i1, i2)
    gsem = (g0, g1)

    cid = lax.axis_index("c")
    sid = lax.axis_index("s")
    wid = cid * NS + sid
    base = wid * E_PER_W

    zero16 = jnp.zeros((16,), jnp.float32)

    def zrow(i, carry):
        for j in range(8):
            rows0[i, pl.ds(j * 16, 16)] = zero16
        return carry

    lax.fori_loop(0, CH, zrow, 0)

    for r in range(8):
        c = sid + NS * r

        @pl.when(c < N_CHUNKS_N)
        def _():
            pltpu.sync_copy(rows0, acc.at[pl.ds(c * CH, CH)])

    plsc.subcore_barrier()

    def idx_issue(k, s):
        off = base + k * CH
        pltpu.async_copy(ei_hbm.at[pl.ds(E + off, CH)], cset[s], isem[s])
        pltpu.async_copy(ei_hbm.at[pl.ds(off, CH)], rset[s], isem[s])
        pltpu.async_copy(ew_hbm.at[pl.ds(off, CH)], wset[s], isem[s])

    def idx_wait(k, s):
        off = base + k * CH
        pltpu.make_async_copy(ei_hbm.at[pl.ds(E + off, CH)], cset[s], isem[s]).wait()
        pltpu.make_async_copy(ei_hbm.at[pl.ds(off, CH)], rset[s], isem[s]).wait()
        pltpu.make_async_copy(ew_hbm.at[pl.ds(off, CH)], wset[s], isem[s]).wait()

    def mul(buf, wv):
        def grp(g, mc):
            w16 = wv[pl.ds(g * 16, 16)]
            for e in range(16):
                ws = jnp.full((16,), w16[e], jnp.float32)
                r_i = g * 16 + e
                for j in range(8):
                    sl = pl.ds(j * 16, 16)
                    buf[r_i, sl] = buf[r_i, sl] * ws
            return mc

        lax.fori_loop(0, CH // 16, grp, 0)

    def step(k, s, rb, do_next_gather, do_idx):
        if do_next_gather:
            idx_wait(k + 1, (s + 1) % 3)
            pltpu.async_copy(h_hbm.at[cset[(s + 1) % 3]], rowsb[1 - rb],
                             gsem[1 - rb])
        pltpu.make_async_copy(h_hbm.at[cset[s]], rowsb[rb], gsem[rb]).wait()
        mul(rowsb[rb], wset[s])
        pltpu.sync_copy(rowsb[rb], acc.at[rset[s]], add=True)
        if do_idx:
            idx_issue(k + 3, s)

    pltpu.sync_copy(ei_hbm.at[pl.ds(E + base, CH)], c0)
    pltpu.sync_copy(ei_hbm.at[pl.ds(base, CH)], r0)
    pltpu.sync_copy(ew_hbm.at[pl.ds(base, CH)], w0)
    pltpu.async_copy(h_hbm.at[c0], rows0, g0)
    idx_issue(1, 1)
    idx_issue(2, 2)

    def six_body(i, carry):
        k = 6 * i
        for off in range(6):
            step(k + off, off % 3, off % 2, True, True)
        return carry

    lax.fori_loop(0, (N_CHUNKS_E - 5) // 6, six_body, 0)

    for k in range((N_CHUNKS_E // 6) * 6, N_CHUNKS_E):
        step(k, k % 3, k % 2, k < N_CHUNKS_E - 1, k + 3 < N_CHUNKS_E)

    for r in range(4):
        c = wid + NW * r

        @pl.when(c < N_CHUNKS_N)
        def _():
            pltpu.sync_copy(prev_hbm.at[pl.ds(c * CH, CH)], c0)
            pltpu.async_copy(h_hbm.at[c0], rows0, psem).wait()
            pltpu.sync_copy(rows0, prevout_hbm.at[pl.ds(c * CH, CH)])

    plsc.subcore_barrier()

    for r in range(8):
        c = sid + NS * r

        @pl.when(c < N_CHUNKS_N)
        def _():
            pltpu.sync_copy(acc.at[pl.ds(c * CH, CH)], rows0)
            pltpu.sync_copy(rows0, partial_hbm.at[cid, pl.ds(c * CH, CH)])


_sc_aggregate = functools.partial(
    pl.kernel,
    out_type=[
        jax.ShapeDtypeStruct((NC, N, D), jnp.float32),
        jax.ShapeDtypeStruct((N, D), jnp.float32),
    ],
    mesh=plsc.VectorSubcoreMesh(
        core_axis_name="c", subcore_axis_name="s", num_cores=NC, num_subcores=NS
    ),
    scratch_types=[
        pltpu.VMEM((CH,), jnp.int32),
        pltpu.VMEM((CH,), jnp.int32),
        pltpu.VMEM((CH,), jnp.int32),
        pltpu.VMEM((CH,), jnp.int32),
        pltpu.VMEM((CH,), jnp.int32),
        pltpu.VMEM((CH,), jnp.int32),
        pltpu.VMEM((CH,), jnp.float32),
        pltpu.VMEM((CH,), jnp.float32),
        pltpu.VMEM((CH,), jnp.float32),
        pltpu.VMEM((CH, D), jnp.float32),
        pltpu.VMEM((CH, D), jnp.float32),
        pltpu.VMEM_SHARED((N, D), jnp.float32),
        pltpu.SemaphoreType.DMA,
        pltpu.SemaphoreType.DMA,
        pltpu.SemaphoreType.DMA,
        pltpu.SemaphoreType.DMA,
        pltpu.SemaphoreType.DMA,
        pltpu.SemaphoreType.DMA,
    ],
)(_sc_body)


def _combine_body(prev_ref, p_ref, out_ref):
    out_ref[:, :D] = prev_ref[...]
    out_ref[:, D:] = p_ref[0] + p_ref[1]


def _combine(prev, partial):
    grid = 10
    blk = N // grid
    return pl.pallas_call(
        _combine_body,
        grid=(grid,),
        in_specs=[
            pl.BlockSpec((blk, D), lambda i: (i, 0)),
            pl.BlockSpec((NC, blk, D), lambda i: (0, i, 0)),
        ],
        out_specs=pl.BlockSpec((blk, 2 * D), lambda i: (i, 0)),
        out_shape=jax.ShapeDtypeStruct((N, 2 * D), jnp.float32),
    )(prev, partial)


def kernel(x, edge_index, edge_weight, previous_index, W, b):
    h = _linear(x, W.T, b.reshape(1, D))
    ei_flat = edge_index.reshape(2 * E)
    partial, prevout = _sc_aggregate(h, ei_flat, edge_weight, previous_index)
    return _combine(prevout, partial)


# speedup vs baseline: 11.8223x; 1.1069x over previous
import functools

import jax
import jax.numpy as jnp
from jax import lax
from jax.experimental import pallas as pl
from jax.experimental.pallas import tpu as pltpu
from jax.experimental.pallas import tpu_sc as plsc

N = 10000
E = 320000
D = 128

NC = 2
NS = 16
NW = NC * NS

CH = 80
E_PER_W = E // NW
N_CHUNKS_E = E_PER_W // CH
N_CHUNKS_N = N // CH


def _linear_body(x_ref, wt_ref, b_ref, out_ref):
    out_ref[...] = (
        jnp.dot(x_ref[...], wt_ref[...], preferred_element_type=jnp.float32)
        + b_ref[...]
    )


def _linear(x, wt, b2d):
    grid = 10
    blk = N // grid
    return pl.pallas_call(
        _linear_body,
        grid=(grid,),
        in_specs=[
            pl.BlockSpec((blk, D), lambda i: (i, 0)),
            pl.BlockSpec((D, D), lambda i: (0, 0)),
            pl.BlockSpec((1, D), lambda i: (0, 0)),
        ],
        out_specs=pl.BlockSpec((blk, D), lambda i: (i, 0)),
        out_shape=jax.ShapeDtypeStruct((N, D), jnp.float32),
    )(x, wt, b2d)


def _sc_body(h_hbm, ei_hbm, ew_hbm, prev_hbm,
             partial_hbm, prevout_hbm,
             c0, c1, c2, w0, w1, w2,
             r0, r1, r2, r3, r4, r5,
             rows0, rows1, rows2, acc,
             i0, i1, i2, i3, i4, i5,
             g0, g1, g2, s0, s1, s2, psem):
    cset = (c0, c1, c2)
    wset = (w0, w1, w2)
    rset = (r0, r1, r2, r3, r4, r5)
    rowsb = (rows0, rows1, rows2)
    isem = (i0, i1, i2, i3, i4, i5)
    gsem = (g0, g1, g2)
    ssem = (s0, s1, s2)

    cid = lax.axis_index("c")
    sid = lax.axis_index("s")
    wid = cid * NS + sid
    base = wid * E_PER_W

    zero16 = jnp.zeros((16,), jnp.float32)

    def zrow(i, carry):
        for j in range(8):
            rows0[i, pl.ds(j * 16, 16)] = zero16
        return carry

    lax.fori_loop(0, CH, zrow, 0)

    for r in range(8):
        c = sid + NS * r

        @pl.when(c < N_CHUNKS_N)
        def _():
            pltpu.sync_copy(rows0, acc.at[pl.ds(c * CH, CH)])

    plsc.subcore_barrier()

    def idx_copies(k_static_mod3, k_static_mod6, off):
        return (
            (ei_hbm.at[pl.ds(E + off, CH)], cset[k_static_mod3]),
            (ei_hbm.at[pl.ds(off, CH)], rset[k_static_mod6]),
            (ew_hbm.at[pl.ds(off, CH)], wset[k_static_mod3]),
        )

    def idx_start(k, m3, m6):
        off = base + k * CH
        for src, dst in idx_copies(m3, m6, off):
            pltpu.async_copy(src, dst, isem[m6])

    def idx_wait(k, m3, m6):
        off = base + k * CH
        for src, dst in idx_copies(m3, m6, off):
            pltpu.make_async_copy(src, dst, isem[m6]).wait()

    def mul(buf, wv):
        def grp(g, mc):
            w16 = wv[pl.ds(g * 16, 16)]
            for e in range(16):
                ws = jnp.full((16,), w16[e], jnp.float32)
                r_i = g * 16 + e
                for j in range(8):
                    sl = pl.ds(j * 16, 16)
                    buf[r_i, sl] = buf[r_i, sl] * ws
            return mc

        lax.fori_loop(0, CH // 16, grp, 0)

    def step(k, m3, m6, do_next_gather, do_idx, guard_sc_wait):
        nm3 = (m3 + 1) % 3
        nm6 = (m6 + 1) % 6
        if do_next_gather:
            idx_wait(k + 1, nm3, nm6)

        def sc_wait():
            pltpu.make_async_copy(
                rowsb[nm3], acc.at[rset[(m6 + 4) % 6]], ssem[nm3]
            ).wait()

        if guard_sc_wait:

            @pl.when(k >= 2)
            def _():
                sc_wait()
        else:
            sc_wait()
        if do_next_gather:
            pltpu.async_copy(h_hbm.at[cset[nm3]], rowsb[nm3], gsem[nm3])
        pltpu.make_async_copy(h_hbm.at[cset[m3]], rowsb[m3], gsem[m3]).wait()
        mul(rowsb[m3], wset[m3])
        pltpu.async_copy(rowsb[m3], acc.at[rset[m6]], add=True, sem=ssem[m3])
        if do_idx:
            idx_start(k + 3, (m3 + 3) % 3, (m6 + 3) % 6)

    pltpu.sync_copy(ei_hbm.at[pl.ds(E + base, CH)], c0)
    pltpu.sync_copy(ei_hbm.at[pl.ds(base, CH)], r0)
    pltpu.sync_copy(ew_hbm.at[pl.ds(base, CH)], w0)
    pltpu.async_copy(h_hbm.at[c0], rows0, g0)
    idx_start(1, 1, 1)
    idx_start(2, 2, 2)

    def six_body(i, carry):
        k = 6 * i
        for off in range(6):
            step(k + off, off % 3, off % 6, True, True, True)
        return carry

    lax.fori_loop(0, (N_CHUNKS_E - 5) // 6, six_body, 0)

    for k in range((N_CHUNKS_E // 6) * 6, N_CHUNKS_E):
        step(k, k % 3, k % 6, k < N_CHUNKS_E - 1, k + 3 < N_CHUNKS_E,
             False)

    k_last = N_CHUNKS_E - 1
    for k in (k_last - 1, k_last):
        pltpu.make_async_copy(
            rowsb[k % 3], acc.at[rset[k % 6]], ssem[k % 3]
        ).wait()

    for r in range(4):
        c = wid + NW * r

        @pl.when(c < N_CHUNKS_N)
        def _():
            pltpu.sync_copy(prev_hbm.at[pl.ds(c * CH, CH)], c0)
            pltpu.async_copy(h_hbm.at[c0], rows0, psem).wait()
            pltpu.sync_copy(rows0, prevout_hbm.at[pl.ds(c * CH, CH)])

    plsc.subcore_barrier()

    for r in range(8):
        c = sid + NS * r

        @pl.when(c < N_CHUNKS_N)
        def _():
            pltpu.sync_copy(acc.at[pl.ds(c * CH, CH)], rows0)
            pltpu.sync_copy(rows0, partial_hbm.at[cid, pl.ds(c * CH, CH)])


_sc_aggregate = functools.partial(
    pl.kernel,
    out_type=[
        jax.ShapeDtypeStruct((NC, N, D), jnp.float32),
        jax.ShapeDtypeStruct((N, D), jnp.float32),
    ],
    mesh=plsc.VectorSubcoreMesh(
        core_axis_name="c", subcore_axis_name="s", num_cores=NC, num_subcores=NS
    ),
    scratch_types=[
        pltpu.VMEM((CH,), jnp.int32),
        pltpu.VMEM((CH,), jnp.int32),
        pltpu.VMEM((CH,), jnp.int32),
        pltpu.VMEM((CH,), jnp.float32),
        pltpu.VMEM((CH,), jnp.float32),
        pltpu.VMEM((CH,), jnp.float32),
        pltpu.VMEM((CH,), jnp.int32),
        pltpu.VMEM((CH,), jnp.int32),
        pltpu.VMEM((CH,), jnp.int32),
        pltpu.VMEM((CH,), jnp.int32),
        pltpu.VMEM((CH,), jnp.int32),
        pltpu.VMEM((CH,), jnp.int32),
        pltpu.VMEM((CH, D), jnp.float32),
        pltpu.VMEM((CH, D), jnp.float32),
        pltpu.VMEM((CH, D), jnp.float32),
        pltpu.VMEM_SHARED((N, D), jnp.float32),
        pltpu.SemaphoreType.DMA,
        pltpu.SemaphoreType.DMA,
        pltpu.SemaphoreType.DMA,
        pltpu.SemaphoreType.DMA,
        pltpu.SemaphoreType.DMA,
        pltpu.SemaphoreType.DMA,
        pltpu.SemaphoreType.DMA,
        pltpu.SemaphoreType.DMA,
        pltpu.SemaphoreType.DMA,
        pltpu.SemaphoreType.DMA,
        pltpu.SemaphoreType.DMA,
        pltpu.SemaphoreType.DMA,
        pltpu.SemaphoreType.DMA,
    ],
)(_sc_body)


def _combine_body(prev_ref, p_ref, out_ref):
    out_ref[:, :D] = prev_ref[...]
    out_ref[:, D:] = p_ref[0] + p_ref[1]


def _combine(prev, partial):
    grid = 10
    blk = N // grid
    return pl.pallas_call(
        _combine_body,
        grid=(grid,),
        in_specs=[
            pl.BlockSpec((blk, D), lambda i: (i, 0)),
            pl.BlockSpec((NC, blk, D), lambda i: (0, i, 0)),
        ],
        out_specs=pl.BlockSpec((blk, 2 * D), lambda i: (i, 0)),
        out_shape=jax.ShapeDtypeStruct((N, 2 * D), jnp.float32),
    )(prev, partial)


def kernel(x, edge_index, edge_weight, previous_index, W, b):
    h = _linear(x, W.T, b.reshape(1, D))
    ei_flat = edge_index.reshape(2 * E)
    partial, prevout = _sc_aggregate(h, ei_flat, edge_weight, previous_index)
    return _combine(prevout, partial)


# speedup vs baseline: 12.3009x; 1.0405x over previous
import functools

import jax
import jax.numpy as jnp
from jax import lax
from jax.experimental import pallas as pl
from jax.experimental.pallas import tpu as pltpu
from jax.experimental.pallas import tpu_sc as plsc

N = 10000
E = 320000
D = 128

NC = 2
NS = 16
NW = NC * NS

CH = 80
E_PER_W = E // NW
N_CHUNKS_E = E_PER_W // CH
N_CHUNKS_N = N // CH


def _linear_body(x_ref, w_ref, b_ref, out_ref):
    out_ref[...] = (
        lax.dot_general(
            x_ref[...], w_ref[...], (((1,), (1,)), ((), ())),
            preferred_element_type=jnp.float32,
        )
        + b_ref[...]
    )


def _linear(x, wt, b2d):
    grid = 10
    blk = N // grid
    return pl.pallas_call(
        _linear_body,
        grid=(grid,),
        in_specs=[
            pl.BlockSpec((blk, D), lambda i: (i, 0)),
            pl.BlockSpec((D, D), lambda i: (0, 0)),
            pl.BlockSpec((1, D), lambda i: (0, 0)),
        ],
        out_specs=pl.BlockSpec((blk, D), lambda i: (i, 0)),
        out_shape=jax.ShapeDtypeStruct((N, D), jnp.float32),
    )(x, wt, b2d)


def _sc_body(h_hbm, ei_hbm, ew_hbm, prev_hbm,
             partial_hbm, prevout_hbm,
             c0, c1, c2, w0, w1, w2,
             r0, r1, r2, r3, r4, r5,
             rows0, rows1, rows2, acc,
             i0, i1, i2, i3, i4, i5,
             g0, g1, g2, s0, s1, s2, psem):
    cset = (c0, c1, c2)
    wset = (w0, w1, w2)
    rset = (r0, r1, r2, r3, r4, r5)
    rowsb = (rows0, rows1, rows2)
    isem = (i0, i1, i2, i3, i4, i5)
    gsem = (g0, g1, g2)
    ssem = (s0, s1, s2)

    cid = lax.axis_index("c")
    sid = lax.axis_index("s")
    wid = cid * NS + sid
    base = wid * E_PER_W

    zero16 = jnp.zeros((16,), jnp.float32)

    def zrow(i, carry):
        for j in range(8):
            rows0[i, pl.ds(j * 16, 16)] = zero16
        return carry

    lax.fori_loop(0, CH, zrow, 0)

    for r in range(8):
        c = sid + NS * r

        @pl.when(c < N_CHUNKS_N)
        def _():
            pltpu.sync_copy(rows0, acc.at[pl.ds(c * CH, CH)])

    plsc.subcore_barrier()

    def idx_copies(k_static_mod3, k_static_mod6, off):
        return (
            (ei_hbm.at[pl.ds(E + off, CH)], cset[k_static_mod3]),
            (ei_hbm.at[pl.ds(off, CH)], rset[k_static_mod6]),
            (ew_hbm.at[pl.ds(off, CH)], wset[k_static_mod3]),
        )

    def idx_start(k, m3, m6):
        off = base + k * CH
        for src, dst in idx_copies(m3, m6, off):
            pltpu.async_copy(src, dst, isem[m6])

    def idx_wait(k, m3, m6):
        off = base + k * CH
        for src, dst in idx_copies(m3, m6, off):
            pltpu.make_async_copy(src, dst, isem[m6]).wait()

    def mul(buf, wv):
        def grp(g, mc):
            w16 = wv[pl.ds(g * 16, 16)]
            for e in range(16):
                ws = jnp.full((16,), w16[e], jnp.float32)
                r_i = g * 16 + e
                for j in range(8):
                    sl = pl.ds(j * 16, 16)
                    buf[r_i, sl] = buf[r_i, sl] * ws
            return mc

        lax.fori_loop(0, CH // 16, grp, 0)

    def step(k, m3, m6, do_next_gather, do_idx, guard_sc_wait):
        nm3 = (m3 + 1) % 3
        nm6 = (m6 + 1) % 6
        if do_next_gather:
            idx_wait(k + 1, nm3, nm6)

        def sc_wait():
            pltpu.make_async_copy(
                rowsb[nm3], acc.at[rset[(m6 + 4) % 6]], ssem[nm3]
            ).wait()

        if guard_sc_wait:

            @pl.when(k >= 2)
            def _():
                sc_wait()
        else:
            sc_wait()
        if do_next_gather:
            pltpu.async_copy(h_hbm.at[cset[nm3]], rowsb[nm3], gsem[nm3])
        pltpu.make_async_copy(h_hbm.at[cset[m3]], rowsb[m3], gsem[m3]).wait()
        mul(rowsb[m3], wset[m3])
        pltpu.async_copy(rowsb[m3], acc.at[rset[m6]], add=True, sem=ssem[m3])
        if do_idx:
            idx_start(k + 3, (m3 + 3) % 3, (m6 + 3) % 6)

    pltpu.sync_copy(ei_hbm.at[pl.ds(E + base, CH)], c0)
    pltpu.sync_copy(ei_hbm.at[pl.ds(base, CH)], r0)
    pltpu.sync_copy(ew_hbm.at[pl.ds(base, CH)], w0)
    pltpu.async_copy(h_hbm.at[c0], rows0, g0)
    idx_start(1, 1, 1)
    idx_start(2, 2, 2)

    def six_body(i, carry):
        k = 6 * i
        for off in range(6):
            step(k + off, off % 3, off % 6, True, True, True)
        return carry

    lax.fori_loop(0, (N_CHUNKS_E - 5) // 6, six_body, 0)

    for k in range((N_CHUNKS_E // 6) * 6, N_CHUNKS_E):
        step(k, k % 3, k % 6, k < N_CHUNKS_E - 1, k + 3 < N_CHUNKS_E,
             False)

    k_last = N_CHUNKS_E - 1
    for k in (k_last - 1, k_last):
        pltpu.make_async_copy(
            rowsb[k % 3], acc.at[rset[k % 6]], ssem[k % 3]
        ).wait()

    pidx = (c0, c1, c2, r0)
    for r in range(4):
        c = wid + NW * r

        @pl.when(c < N_CHUNKS_N)
        def _():
            pltpu.async_copy(prev_hbm.at[pl.ds(c * CH, CH)], pidx[r], isem[r])

    def prev_gather_start(r):
        c = wid + NW * r

        @pl.when(c < N_CHUNKS_N)
        def _():
            pltpu.make_async_copy(
                prev_hbm.at[pl.ds(c * CH, CH)], pidx[r], isem[r]
            ).wait()
            pltpu.async_copy(h_hbm.at[pidx[r]], rowsb[r % 3], gsem[r % 3])

    def prev_writeback(r):
        c = wid + NW * r

        @pl.when(c < N_CHUNKS_N)
        def _():
            pltpu.make_async_copy(
                h_hbm.at[pidx[r]], rowsb[r % 3], gsem[r % 3]
            ).wait()
            pltpu.sync_copy(rowsb[r % 3], prevout_hbm.at[pl.ds(c * CH, CH)])

    for r in range(3):
        prev_gather_start(r)
    prev_writeback(0)
    prev_gather_start(3)
    for r in range(1, 4):
        prev_writeback(r)

    plsc.subcore_barrier()

    for r in range(8):
        c = sid + NS * r

        @pl.when(c < N_CHUNKS_N)
        def _():
            pltpu.sync_copy(acc.at[pl.ds(c * CH, CH)],
                            partial_hbm.at[cid, pl.ds(c * CH, CH)])


_sc_aggregate = functools.partial(
    pl.kernel,
    out_type=[
        jax.ShapeDtypeStruct((NC, N, D), jnp.float32),
        jax.ShapeDtypeStruct((N, D), jnp.float32),
    ],
    mesh=plsc.VectorSubcoreMesh(
        core_axis_name="c", subcore_axis_name="s", num_cores=NC, num_subcores=NS
    ),
    scratch_types=[
        pltpu.VMEM((CH,), jnp.int32),
        pltpu.VMEM((CH,), jnp.int32),
        pltpu.VMEM((CH,), jnp.int32),
        pltpu.VMEM((CH,), jnp.float32),
        pltpu.VMEM((CH,), jnp.float32),
        pltpu.VMEM((CH,), jnp.float32),
        pltpu.VMEM((CH,), jnp.int32),
        pltpu.VMEM((CH,), jnp.int32),
        pltpu.VMEM((CH,), jnp.int32),
        pltpu.VMEM((CH,), jnp.int32),
        pltpu.VMEM((CH,), jnp.int32),
        pltpu.VMEM((CH,), jnp.int32),
        pltpu.VMEM((CH, D), jnp.float32),
        pltpu.VMEM((CH, D), jnp.float32),
        pltpu.VMEM((CH, D), jnp.float32),
        pltpu.VMEM_SHARED((N, D), jnp.float32),
        pltpu.SemaphoreType.DMA,
        pltpu.SemaphoreType.DMA,
        pltpu.SemaphoreType.DMA,
        pltpu.SemaphoreType.DMA,
        pltpu.SemaphoreType.DMA,
        pltpu.SemaphoreType.DMA,
        pltpu.SemaphoreType.DMA,
        pltpu.SemaphoreType.DMA,
        pltpu.SemaphoreType.DMA,
        pltpu.SemaphoreType.DMA,
        pltpu.SemaphoreType.DMA,
        pltpu.SemaphoreType.DMA,
        pltpu.SemaphoreType.DMA,
    ],
)(_sc_body)


def _combine_body(prev_ref, p_ref, out_ref):
    out_ref[:, :D] = prev_ref[...]
    out_ref[:, D:] = p_ref[0] + p_ref[1]


def _combine(prev, partial):
    grid = 10
    blk = N // grid
    return pl.pallas_call(
        _combine_body,
        grid=(grid,),
        in_specs=[
            pl.BlockSpec((blk, D), lambda i: (i, 0)),
            pl.BlockSpec((NC, blk, D), lambda i: (0, i, 0)),
        ],
        out_specs=pl.BlockSpec((blk, 2 * D), lambda i: (i, 0)),
        out_shape=jax.ShapeDtypeStruct((N, 2 * D), jnp.float32),
    )(prev, partial)


def kernel(x, edge_index, edge_weight, previous_index, W, b):
    h = _linear(x, W, b.reshape(1, D))
    ei_flat = edge_index.reshape(2 * E)
    partial, prevout = _sc_aggregate(h, ei_flat, edge_weight, previous_index)
    return _combine(prevout, partial)


# speedup vs baseline: 12.4952x; 1.0158x over previous
import functools

import jax
import jax.numpy as jnp
from jax import lax
from jax.experimental import pallas as pl
from jax.experimental.pallas import tpu as pltpu
from jax.experimental.pallas import tpu_sc as plsc

N = 10000
E = 320000
D = 128

NC = 2
NS = 16
NW = NC * NS

CH = 80
E_PER_W = E // NW
N_CHUNKS_E = E_PER_W // CH
N_CHUNKS_N = N // CH


def _linear_body(x_ref, w_ref, b_ref, out_ref):
    out_ref[...] = (
        lax.dot_general(
            x_ref[...], w_ref[...], (((1,), (1,)), ((), ())),
            preferred_element_type=jnp.float32,
        )
        + b_ref[...]
    )


def _linear(x, wt, b2d):
    grid = 10
    blk = N // grid
    return pl.pallas_call(
        _linear_body,
        grid=(grid,),
        in_specs=[
            pl.BlockSpec((blk, D), lambda i: (i, 0)),
            pl.BlockSpec((D, D), lambda i: (0, 0)),
            pl.BlockSpec((1, D), lambda i: (0, 0)),
        ],
        out_specs=pl.BlockSpec((blk, D), lambda i: (i, 0)),
        out_shape=jax.ShapeDtypeStruct((N, D), jnp.float32),
    )(x, wt, b2d)


def _sc_body(h_hbm, ei_hbm, ew_hbm, prev_hbm,
             partial_hbm, prevout_hbm,
             c0, c1, c2, w0, w1, w2,
             r0, r1, r2, r3, r4, r5,
             rows0, rows1, rows2, acc,
             i0, i1, i2, i3, i4, i5,
             g0, g1, g2, s0, s1, s2, psem):
    cset = (c0, c1, c2)
    wset = (w0, w1, w2)
    rset = (r0, r1, r2, r3, r4, r5)
    rowsb = (rows0, rows1, rows2)
    isem = (i0, i1, i2, i3, i4, i5)
    gsem = (g0, g1, g2)
    ssem = (s0, s1, s2)

    cid = lax.axis_index("c")
    sid = lax.axis_index("s")
    wid = cid * NS + sid
    base = wid * E_PER_W

    zero16 = jnp.zeros((16,), jnp.float32)

    def zrow(i, carry):
        for j in range(8):
            rows0[i, pl.ds(j * 16, 16)] = zero16
        return carry

    lax.fori_loop(0, CH, zrow, 0)

    for r in range(8):
        c = sid + NS * r

        @pl.when(c < N_CHUNKS_N)
        def _():
            pltpu.sync_copy(rows0, acc.at[pl.ds(c * CH, CH)])

    plsc.subcore_barrier()

    def idx_copies(k_static_mod3, k_static_mod6, off):
        return (
            (ei_hbm.at[pl.ds(E + off, CH)], cset[k_static_mod3]),
            (ei_hbm.at[pl.ds(off, CH)], rset[k_static_mod6]),
            (ew_hbm.at[pl.ds(off, CH)], wset[k_static_mod3]),
        )

    def idx_start(k, m3, m6):
        off = base + k * CH
        for src, dst in idx_copies(m3, m6, off):
            pltpu.async_copy(src, dst, isem[m6])

    def idx_wait(k, m3, m6):
        off = base + k * CH
        for src, dst in idx_copies(m3, m6, off):
            pltpu.make_async_copy(src, dst, isem[m6]).wait()

    def mul(buf, wv):
        def grp(g, mc):
            w16 = wv[pl.ds(g * 16, 16)]
            for e in range(16):
                ws = jnp.full((16,), w16[e], jnp.float32)
                r_i = g * 16 + e
                for j in range(8):
                    sl = pl.ds(j * 16, 16)
                    buf[r_i, sl] = buf[r_i, sl] * ws
            return mc

        lax.fori_loop(0, CH // 16, grp, 0)

    def step(k, m3, m6, do_next_gather, do_idx, guard_sc_wait):
        nm3 = (m3 + 1) % 3
        nm6 = (m6 + 1) % 6
        if do_next_gather:
            idx_wait(k + 1, nm3, nm6)

        def sc_wait():
            pltpu.make_async_copy(
                rowsb[nm3], acc.at[rset[(m6 + 4) % 6]], ssem[nm3]
            ).wait()

        if guard_sc_wait:

            @pl.when(k >= 2)
            def _():
                sc_wait()
        else:
            sc_wait()
        if do_next_gather:
            pltpu.async_copy(h_hbm.at[cset[nm3]], rowsb[nm3], gsem[nm3])
        pltpu.make_async_copy(h_hbm.at[cset[m3]], rowsb[m3], gsem[m3]).wait()
        mul(rowsb[m3], wset[m3])
        pltpu.async_copy(rowsb[m3], acc.at[rset[m6]], add=True, sem=ssem[m3])
        if do_idx:
            idx_start(k + 3, (m3 + 3) % 3, (m6 + 3) % 6)

    pltpu.sync_copy(ei_hbm.at[pl.ds(E + base, CH)], c0)
    pltpu.sync_copy(ei_hbm.at[pl.ds(base, CH)], r0)
    pltpu.sync_copy(ew_hbm.at[pl.ds(base, CH)], w0)
    pltpu.async_copy(h_hbm.at[c0], rows0, g0)
    idx_start(1, 1, 1)
    idx_start(2, 2, 2)

    def six_body(i, carry):
        k = 6 * i
        for off in range(6):
            step(k + off, off % 3, off % 6, True, True, True)
        return carry

    lax.fori_loop(0, (N_CHUNKS_E - 5) // 6, six_body, 0)

    for k in range((N_CHUNKS_E // 6) * 6, N_CHUNKS_E):
        step(k, k % 3, k % 6, k < N_CHUNKS_E - 1, k + 3 < N_CHUNKS_E,
             False)

    k_last = N_CHUNKS_E - 1
    for k in (k_last - 1, k_last):
        pltpu.make_async_copy(
            rowsb[k % 3], acc.at[rset[k % 6]], ssem[k % 3]
        ).wait()

    pidx = (c0, c1, c2, r0)
    for r in range(4):
        c = wid + NW * r

        @pl.when(c < N_CHUNKS_N)
        def _():
            pltpu.async_copy(prev_hbm.at[pl.ds(c * CH, CH)], pidx[r], isem[r])

    def prev_gather_start(r):
        c = wid + NW * r

        @pl.when(c < N_CHUNKS_N)
        def _():
            pltpu.make_async_copy(
                prev_hbm.at[pl.ds(c * CH, CH)], pidx[r], isem[r]
            ).wait()
            pltpu.async_copy(h_hbm.at[pidx[r]], rowsb[r % 3], gsem[r % 3])

    def prev_writeback(r):
        c = wid + NW * r

        @pl.when(c < N_CHUNKS_N)
        def _():
            pltpu.make_async_copy(
                h_hbm.at[pidx[r]], rowsb[r % 3], gsem[r % 3]
            ).wait()
            pltpu.sync_copy(
                rowsb[r % 3],
                prevout_hbm.at[pl.ds(c * CH, CH), pl.ds(0, D)],
            )

    for r in range(3):
        prev_gather_start(r)
    prev_writeback(0)
    prev_gather_start(3)
    for r in range(1, 4):
        prev_writeback(r)

    plsc.subcore_barrier()

    for r in range(8):
        c = sid + NS * r

        @pl.when(c < N_CHUNKS_N)
        def _():
            pltpu.sync_copy(acc.at[pl.ds(c * CH, CH)],
                            partial_hbm.at[cid, pl.ds(c * CH, CH)])


_sc_aggregate = functools.partial(
    pl.kernel,
    out_type=[
        jax.ShapeDtypeStruct((NC, N, D), jnp.float32),
        jax.ShapeDtypeStruct((N, 2 * D), jnp.float32),
    ],
    mesh=plsc.VectorSubcoreMesh(
        core_axis_name="c", subcore_axis_name="s", num_cores=NC, num_subcores=NS
    ),
    scratch_types=[
        pltpu.VMEM((CH,), jnp.int32),
        pltpu.VMEM((CH,), jnp.int32),
        pltpu.VMEM((CH,), jnp.int32),
        pltpu.VMEM((CH,), jnp.float32),
        pltpu.VMEM((CH,), jnp.float32),
        pltpu.VMEM((CH,), jnp.float32),
        pltpu.VMEM((CH,), jnp.int32),
        pltpu.VMEM((CH,), jnp.int32),
        pltpu.VMEM((CH,), jnp.int32),
        pltpu.VMEM((CH,), jnp.int32),
        pltpu.VMEM((CH,), jnp.int32),
        pltpu.VMEM((CH,), jnp.int32),
        pltpu.VMEM((CH, D), jnp.float32),
        pltpu.VMEM((CH, D), jnp.float32),
        pltpu.VMEM((CH, D), jnp.float32),
        pltpu.VMEM_SHARED((N, D), jnp.float32),
        pltpu.SemaphoreType.DMA,
        pltpu.SemaphoreType.DMA,
        pltpu.SemaphoreType.DMA,
        pltpu.SemaphoreType.DMA,
        pltpu.SemaphoreType.DMA,
        pltpu.SemaphoreType.DMA,
        pltpu.SemaphoreType.DMA,
        pltpu.SemaphoreType.DMA,
        pltpu.SemaphoreType.DMA,
        pltpu.SemaphoreType.DMA,
        pltpu.SemaphoreType.DMA,
        pltpu.SemaphoreType.DMA,
        pltpu.SemaphoreType.DMA,
    ],
)(_sc_body)


def _combine_body(oi_ref, p_ref, out_ref):
    out_ref[...] = p_ref[0] + p_ref[1]


def _combine(outbuf, partial):
    grid = 10
    blk = N // grid
    return pl.pallas_call(
        _combine_body,
        grid=(grid,),
        in_specs=[
            pl.BlockSpec(memory_space=pl.ANY),
            pl.BlockSpec((NC, blk, D), lambda i: (0, i, 0)),
        ],
        out_specs=pl.BlockSpec((blk, D), lambda i: (i, 1)),
        out_shape=jax.ShapeDtypeStruct((N, 2 * D), jnp.float32),
        input_output_aliases={0: 0},
    )(outbuf, partial)


def kernel(x, edge_index, edge_weight, previous_index, W, b):
    h = _linear(x, W, b.reshape(1, D))
    ei_flat = edge_index.reshape(2 * E)
    partial, outbuf = _sc_aggregate(h, ei_flat, edge_weight, previous_index)
    return _combine(outbuf, partial)


# speedup vs baseline: 12.5675x; 1.0058x over previous
import functools

import jax
import jax.numpy as jnp
from jax import lax
from jax.experimental import pallas as pl
from jax.experimental.pallas import tpu as pltpu
from jax.experimental.pallas import tpu_sc as plsc

N = 10000
E = 320000
D = 128

NC = 2
NS = 16
NW = NC * NS

CH = 80
E_PER_W = E // NW
N_CHUNKS_E = E_PER_W // CH
N_CHUNKS_N = N // CH


def _linear_body(x_ref, w_ref, b_ref, out_ref):
    out_ref[...] = (
        lax.dot_general(
            x_ref[...], w_ref[...], (((1,), (1,)), ((), ())),
            preferred_element_type=jnp.float32,
        )
        + b_ref[...]
    )


def _linear(x, wt, b2d):
    grid = 10
    blk = N // grid
    return pl.pallas_call(
        _linear_body,
        grid=(grid,),
        in_specs=[
            pl.BlockSpec((blk, D), lambda i: (i, 0)),
            pl.BlockSpec((D, D), lambda i: (0, 0)),
            pl.BlockSpec((1, D), lambda i: (0, 0)),
        ],
        out_specs=pl.BlockSpec((blk, D), lambda i: (i, 0)),
        out_shape=jax.ShapeDtypeStruct((N, D), jnp.float32),
    )(x, wt, b2d)


def _sc_body(h_hbm, ei_hbm, ew_hbm, prev_hbm,
             partial_hbm, prevout_hbm,
             c0, c1, c2, w0, w1, w2,
             r0, r1, r2, r3, r4, r5,
             rows0, rows1, rows2, acc,
             i0, i1, i2, i3, i4, i5,
             g0, g1, g2, s0, s1, s2, psem):
    cset = (c0, c1, c2)
    wset = (w0, w1, w2)
    rset = (r0, r1, r2, r3, r4, r5)
    rowsb = (rows0, rows1, rows2)
    isem = (i0, i1, i2, i3, i4, i5)
    gsem = (g0, g1, g2)
    ssem = (s0, s1, s2)

    cid = lax.axis_index("c")
    sid = lax.axis_index("s")
    wid = cid * NS + sid
    base = wid * E_PER_W

    zero16 = jnp.zeros((16,), jnp.float32)

    def zrow(i, carry):
        for j in range(8):
            rows0[i, pl.ds(j * 16, 16)] = zero16
        return carry

    lax.fori_loop(0, CH, zrow, 0)

    for r in range(8):
        c = sid + NS * r

        @pl.when(c < N_CHUNKS_N)
        def _():
            pltpu.async_copy(rows0, acc.at[pl.ds(c * CH, CH)], psem)

    def idx_copies(k_static_mod3, k_static_mod6, off):
        return (
            (ei_hbm.at[pl.ds(E + off, CH)], cset[k_static_mod3]),
            (ei_hbm.at[pl.ds(off, CH)], rset[k_static_mod6]),
            (ew_hbm.at[pl.ds(off, CH)], wset[k_static_mod3]),
        )

    def idx_start(k, m3, m6):
        off = base + k * CH
        for src, dst in idx_copies(m3, m6, off):
            pltpu.async_copy(src, dst, isem[m6])

    def idx_wait(k, m3, m6):
        off = base + k * CH
        for src, dst in idx_copies(m3, m6, off):
            pltpu.make_async_copy(src, dst, isem[m6]).wait()

    def mul(buf, wv):
        def grp(g, mc):
            w16 = wv[pl.ds(g * 16, 16)]
            for e in range(16):
                ws = jnp.full((16,), w16[e], jnp.float32)
                r_i = g * 16 + e
                for j in range(8):
                    sl = pl.ds(j * 16, 16)
                    buf[r_i, sl] = buf[r_i, sl] * ws
            return mc

        lax.fori_loop(0, CH // 16, grp, 0)

    def step(k, m3, m6, do_next_gather, do_idx, guard_sc_wait):
        nm3 = (m3 + 1) % 3
        nm6 = (m6 + 1) % 6
        if do_next_gather:
            idx_wait(k + 1, nm3, nm6)

        def sc_wait():
            pltpu.make_async_copy(
                rowsb[nm3], acc.at[rset[(m6 + 4) % 6]], ssem[nm3]
            ).wait()

        if guard_sc_wait:

            @pl.when(k >= 2)
            def _():
                sc_wait()
        else:
            sc_wait()
        if do_next_gather:
            pltpu.async_copy(h_hbm.at[cset[nm3]], rowsb[nm3], gsem[nm3])
        pltpu.make_async_copy(h_hbm.at[cset[m3]], rowsb[m3], gsem[m3]).wait()
        mul(rowsb[m3], wset[m3])
        pltpu.async_copy(rowsb[m3], acc.at[rset[m6]], add=True, sem=ssem[m3])
        if do_idx:
            idx_start(k + 3, (m3 + 3) % 3, (m6 + 3) % 6)

    pltpu.sync_copy(ei_hbm.at[pl.ds(E + base, CH)], c0)
    pltpu.sync_copy(ei_hbm.at[pl.ds(base, CH)], r0)
    pltpu.sync_copy(ew_hbm.at[pl.ds(base, CH)], w0)
    idx_start(1, 1, 1)
    idx_start(2, 2, 2)
    for r in range(8):
        c = sid + NS * r

        @pl.when(c < N_CHUNKS_N)
        def _():
            pltpu.make_async_copy(rows0, acc.at[pl.ds(c * CH, CH)], psem).wait()

    plsc.subcore_barrier()
    pltpu.async_copy(h_hbm.at[c0], rows0, g0)

    def six_body(i, carry):
        k = 6 * i
        for off in range(6):
            step(k + off, off % 3, off % 6, True, True, True)
        return carry

    lax.fori_loop(0, (N_CHUNKS_E - 5) // 6, six_body, 0)

    for k in range((N_CHUNKS_E // 6) * 6, N_CHUNKS_E):
        step(k, k % 3, k % 6, k < N_CHUNKS_E - 1, k + 3 < N_CHUNKS_E,
             False)

    k_last = N_CHUNKS_E - 1
    for k in (k_last - 1, k_last):
        pltpu.make_async_copy(
            rowsb[k % 3], acc.at[rset[k % 6]], ssem[k % 3]
        ).wait()

    pidx = (c0, c1, c2, r0)
    for r in range(4):
        c = wid + NW * r

        @pl.when(c < N_CHUNKS_N)
        def _():
            pltpu.async_copy(prev_hbm.at[pl.ds(c * CH, CH)], pidx[r], isem[r])

    def prev_gather_start(r):
        c = wid + NW * r

        @pl.when(c < N_CHUNKS_N)
        def _():
            pltpu.make_async_copy(
                prev_hbm.at[pl.ds(c * CH, CH)], pidx[r], isem[r]
            ).wait()
            pltpu.async_copy(h_hbm.at[pidx[r]], rowsb[r % 3], gsem[r % 3])

    def prev_writeback(r):
        c = wid + NW * r

        @pl.when(c < N_CHUNKS_N)
        def _():
            pltpu.make_async_copy(
                h_hbm.at[pidx[r]], rowsb[r % 3], gsem[r % 3]
            ).wait()
            pltpu.sync_copy(
                rowsb[r % 3],
                prevout_hbm.at[pl.ds(c * CH, CH), pl.ds(0, D)],
            )

    for r in range(3):
        prev_gather_start(r)
    prev_writeback(0)
    prev_gather_start(3)
    for r in range(1, 4):
        prev_writeback(r)

    plsc.subcore_barrier()

    for r in range(8):
        c = sid + NS * r

        @pl.when(c < N_CHUNKS_N)
        def _():
            pltpu.sync_copy(acc.at[pl.ds(c * CH, CH)],
                            partial_hbm.at[cid, pl.ds(c * CH, CH)])


_sc_aggregate = functools.partial(
    pl.kernel,
    out_type=[
        jax.ShapeDtypeStruct((NC, N, D), jnp.float32),
        jax.ShapeDtypeStruct((N, 2 * D), jnp.float32),
    ],
    mesh=plsc.VectorSubcoreMesh(
        core_axis_name="c", subcore_axis_name="s", num_cores=NC, num_subcores=NS
    ),
    scratch_types=[
        pltpu.VMEM((CH,), jnp.int32),
        pltpu.VMEM((CH,), jnp.int32),
        pltpu.VMEM((CH,), jnp.int32),
        pltpu.VMEM((CH,), jnp.float32),
        pltpu.VMEM((CH,), jnp.float32),
        pltpu.VMEM((CH,), jnp.float32),
        pltpu.VMEM((CH,), jnp.int32),
        pltpu.VMEM((CH,), jnp.int32),
        pltpu.VMEM((CH,), jnp.int32),
        pltpu.VMEM((CH,), jnp.int32),
        pltpu.VMEM((CH,), jnp.int32),
        pltpu.VMEM((CH,), jnp.int32),
        pltpu.VMEM((CH, D), jnp.float32),
        pltpu.VMEM((CH, D), jnp.float32),
        pltpu.VMEM((CH, D), jnp.float32),
        pltpu.VMEM_SHARED((N, D), jnp.float32),
        pltpu.SemaphoreType.DMA,
        pltpu.SemaphoreType.DMA,
        pltpu.SemaphoreType.DMA,
        pltpu.SemaphoreType.DMA,
        pltpu.SemaphoreType.DMA,
        pltpu.SemaphoreType.DMA,
        pltpu.SemaphoreType.DMA,
        pltpu.SemaphoreType.DMA,
        pltpu.SemaphoreType.DMA,
        pltpu.SemaphoreType.DMA,
        pltpu.SemaphoreType.DMA,
        pltpu.SemaphoreType.DMA,
        pltpu.SemaphoreType.DMA,
    ],
)(_sc_body)


def _combine_body(oi_ref, p_ref, out_ref):
    out_ref[...] = p_ref[0] + p_ref[1]


def _combine(outbuf, partial):
    grid = 10
    blk = N // grid
    return pl.pallas_call(
        _combine_body,
        grid=(grid,),
        in_specs=[
            pl.BlockSpec(memory_space=pl.ANY),
            pl.BlockSpec((NC, blk, D), lambda i: (0, i, 0)),
        ],
        out_specs=pl.BlockSpec((blk, D), lambda i: (i, 1)),
        out_shape=jax.ShapeDtypeStruct((N, 2 * D), jnp.float32),
        input_output_aliases={0: 0},
    )(outbuf, partial)


def kernel(x, edge_index, edge_weight, previous_index, W, b):
    h = _linear(x, W, b.reshape(1, D))
    ei_flat = edge_index.reshape(2 * E)
    partial, outbuf = _sc_aggregate(h, ei_flat, edge_weight, previous_index)
    return _combine(outbuf, partial)


# speedup vs baseline: 12.5762x; 1.0007x over previous
import functools

import jax
import jax.numpy as jnp
from jax import lax
from jax.experimental import pallas as pl
from jax.experimental.pallas import tpu as pltpu
from jax.experimental.pallas import tpu_sc as plsc

N = 10000
E = 320000
D = 128

NC = 2
NS = 16
NW = NC * NS

CH = 80
E_PER_W = E // NW
N_CHUNKS_E = E_PER_W // CH
N_CHUNKS_N = N // CH


def _linear_body(x_ref, w_ref, b_ref, out_ref):
    out_ref[...] = (
        lax.dot_general(
            x_ref[...], w_ref[...], (((1,), (1,)), ((), ())),
            preferred_element_type=jnp.float32,
        )
        + b_ref[...]
    )


def _linear(x, wt, b2d):
    grid = 10
    blk = N // grid
    return pl.pallas_call(
        _linear_body,
        grid=(grid,),
        in_specs=[
            pl.BlockSpec((blk, D), lambda i: (i, 0)),
            pl.BlockSpec((D, D), lambda i: (0, 0)),
            pl.BlockSpec((1, D), lambda i: (0, 0)),
        ],
        out_specs=pl.BlockSpec((blk, D), lambda i: (i, 0)),
        out_shape=jax.ShapeDtypeStruct((N, D), jnp.float32),
    )(x, wt, b2d)


def _sc_body(h_hbm, ei_hbm, ew_hbm, prev_hbm,
             partial_hbm, prevout_hbm,
             c0, c1, c2, w0, w1, w2,
             r0, r1, r2, r3, r4, r5,
             rows0, rows1, rows2, acc,
             i0, i1, i2, i3, i4, i5,
             g0, g1, g2, s0, s1, s2, psem):
    cset = (c0, c1, c2)
    wset = (w0, w1, w2)
    rset = (r0, r1, r2, r3, r4, r5)
    rowsb = (rows0, rows1, rows2)
    isem = (i0, i1, i2, i3, i4, i5)
    gsem = (g0, g1, g2)
    ssem = (s0, s1, s2)

    cid = lax.axis_index("c")
    sid = lax.axis_index("s")
    wid = cid * NS + sid
    base = wid * E_PER_W

    zero16 = jnp.zeros((16,), jnp.float32)

    def zrow(i, carry):
        for j in range(8):
            rows0[i, pl.ds(j * 16, 16)] = zero16
        return carry

    lax.fori_loop(0, CH, zrow, 0)

    for r in range(8):
        c = sid + NS * r

        @pl.when(c < N_CHUNKS_N)
        def _():
            pltpu.async_copy(rows0, acc.at[pl.ds(c * CH, CH)], psem)

    def idx_copies(k_static_mod3, k_static_mod6, off):
        return (
            (ei_hbm.at[pl.ds(E + off, CH)], cset[k_static_mod3]),
            (ei_hbm.at[pl.ds(off, CH)], rset[k_static_mod6]),
            (ew_hbm.at[pl.ds(off, CH)], wset[k_static_mod3]),
        )

    def idx_start(k, m3, m6):
        off = base + k * CH
        for src, dst in idx_copies(m3, m6, off):
            pltpu.async_copy(src, dst, isem[m6])

    def idx_wait(k, m3, m6):
        off = base + k * CH
        for src, dst in idx_copies(m3, m6, off):
            pltpu.make_async_copy(src, dst, isem[m6]).wait()

    gdn = lax.GatherDimensionNumbers(
        offset_dims=(), collapsed_slice_dims=(0,), start_index_map=(0,)
    )

    def mul(buf, wv):
        def grp(g, mc):
            w16 = wv[pl.ds(g * 16, 16)]
            for e in range(16):
                ws = lax.gather(
                    w16, jnp.full((16, 1), e, jnp.int32), gdn, (1,),
                    mode=lax.GatherScatterMode.PROMISE_IN_BOUNDS,
                )
                r_i = g * 16 + e
                for j in range(8):
                    sl = pl.ds(j * 16, 16)
                    buf[r_i, sl] = buf[r_i, sl] * ws
            return mc

        lax.fori_loop(0, CH // 16, grp, 0)

    def step(k, m3, m6, do_next_gather, do_idx, guard_sc_wait):
        nm3 = (m3 + 1) % 3
        nm6 = (m6 + 1) % 6
        if do_next_gather:
            idx_wait(k + 1, nm3, nm6)

        def sc_wait():
            pltpu.make_async_copy(
                rowsb[nm3], acc.at[rset[(m6 + 4) % 6]], ssem[nm3]
            ).wait()

        if guard_sc_wait:

            @pl.when(k >= 2)
            def _():
                sc_wait()
        else:
            sc_wait()
        if do_next_gather:
            pltpu.async_copy(h_hbm.at[cset[nm3]], rowsb[nm3], gsem[nm3])
        pltpu.make_async_copy(h_hbm.at[cset[m3]], rowsb[m3], gsem[m3]).wait()
        mul(rowsb[m3], wset[m3])
        pltpu.async_copy(rowsb[m3], acc.at[rset[m6]], add=True, sem=ssem[m3])
        if do_idx:
            idx_start(k + 3, (m3 + 3) % 3, (m6 + 3) % 6)

    pltpu.sync_copy(ei_hbm.at[pl.ds(E + base, CH)], c0)
    pltpu.sync_copy(ei_hbm.at[pl.ds(base, CH)], r0)
    pltpu.sync_copy(ew_hbm.at[pl.ds(base, CH)], w0)
    idx_start(1, 1, 1)
    idx_start(2, 2, 2)
    for r in range(8):
        c = sid + NS * r

        @pl.when(c < N_CHUNKS_N)
        def _():
            pltpu.make_async_copy(rows0, acc.at[pl.ds(c * CH, CH)], psem).wait()

    plsc.subcore_barrier()
    pltpu.async_copy(h_hbm.at[c0], rows0, g0)

    def six_body(i, carry):
        k = 6 * i
        for off in range(6):
            step(k + off, off % 3, off % 6, True, True, True)
        return carry

    lax.fori_loop(0, (N_CHUNKS_E - 5) // 6, six_body, 0)

    for k in range((N_CHUNKS_E // 6) * 6, N_CHUNKS_E):
        step(k, k % 3, k % 6, k < N_CHUNKS_E - 1, k + 3 < N_CHUNKS_E,
             False)

    k_last = N_CHUNKS_E - 1
    for k in (k_last - 1, k_last):
        pltpu.make_async_copy(
            rowsb[k % 3], acc.at[rset[k % 6]], ssem[k % 3]
        ).wait()

    pidx = (c0, c1, c2, r0)
    for r in range(4):
        c = wid + NW * r

        @pl.when(c < N_CHUNKS_N)
        def _():
            pltpu.async_copy(prev_hbm.at[pl.ds(c * CH, CH)], pidx[r], isem[r])

    def prev_gather_start(r):
        c = wid + NW * r

        @pl.when(c < N_CHUNKS_N)
        def _():
            pltpu.make_async_copy(
                prev_hbm.at[pl.ds(c * CH, CH)], pidx[r], isem[r]
            ).wait()
            pltpu.async_copy(h_hbm.at[pidx[r]], rowsb[r % 3], gsem[r % 3])

    def prev_writeback(r):
        c = wid + NW * r

        @pl.when(c < N_CHUNKS_N)
        def _():
            pltpu.make_async_copy(
                h_hbm.at[pidx[r]], rowsb[r % 3], gsem[r % 3]
            ).wait()
            pltpu.sync_copy(
                rowsb[r % 3],
                prevout_hbm.at[pl.ds(c * CH, CH), pl.ds(0, D)],
            )

    for r in range(3):
        prev_gather_start(r)
    prev_writeback(0)
    prev_gather_start(3)
    for r in range(1, 4):
        prev_writeback(r)

    plsc.subcore_barrier()

    for r in range(8):
        c = sid + NS * r

        @pl.when(c < N_CHUNKS_N)
        def _():
            pltpu.sync_copy(acc.at[pl.ds(c * CH, CH)],
                            partial_hbm.at[cid, pl.ds(c * CH, CH)])


_sc_aggregate = functools.partial(
    pl.kernel,
    out_type=[
        jax.ShapeDtypeStruct((NC, N, D), jnp.float32),
        jax.ShapeDtypeStruct((N, 2 * D), jnp.float32),
    ],
    mesh=plsc.VectorSubcoreMesh(
        core_axis_name="c", subcore_axis_name="s", num_cores=NC, num_subcores=NS
    ),
    scratch_types=[
        pltpu.VMEM((CH,), jnp.int32),
        pltpu.VMEM((CH,), jnp.int32),
        pltpu.VMEM((CH,), jnp.int32),
        pltpu.VMEM((CH,), jnp.float32),
        pltpu.VMEM((CH,), jnp.float32),
        pltpu.VMEM((CH,), jnp.float32),
        pltpu.VMEM((CH,), jnp.int32),
        pltpu.VMEM((CH,), jnp.int32),
        pltpu.VMEM((CH,), jnp.int32),
        pltpu.VMEM((CH,), jnp.int32),
        pltpu.VMEM((CH,), jnp.int32),
        pltpu.VMEM((CH,), jnp.int32),
        pltpu.VMEM((CH, D), jnp.float32),
        pltpu.VMEM((CH, D), jnp.float32),
        pltpu.VMEM((CH, D), jnp.float32),
        pltpu.VMEM_SHARED((N, D), jnp.float32),
        pltpu.SemaphoreType.DMA,
        pltpu.SemaphoreType.DMA,
        pltpu.SemaphoreType.DMA,
        pltpu.SemaphoreType.DMA,
        pltpu.SemaphoreType.DMA,
        pltpu.SemaphoreType.DMA,
        pltpu.SemaphoreType.DMA,
        pltpu.SemaphoreType.DMA,
        pltpu.SemaphoreType.DMA,
        pltpu.SemaphoreType.DMA,
        pltpu.SemaphoreType.DMA,
        pltpu.SemaphoreType.DMA,
        pltpu.SemaphoreType.DMA,
    ],
)(_sc_body)


def _combine_body(oi_ref, p_ref, out_ref):
    out_ref[...] = p_ref[0] + p_ref[1]


def _combine(outbuf, partial):
    grid = 10
    blk = N // grid
    return pl.pallas_call(
        _combine_body,
        grid=(grid,),
        in_specs=[
            pl.BlockSpec(memory_space=pl.ANY),
            pl.BlockSpec((NC, blk, D), lambda i: (0, i, 0)),
        ],
        out_specs=pl.BlockSpec((blk, D), lambda i: (i, 1)),
        out_shape=jax.ShapeDtypeStruct((N, 2 * D), jnp.float32),
        input_output_aliases={0: 0},
    )(outbuf, partial)


def kernel(x, edge_index, edge_weight, previous_index, W, b):
    h = _linear(x, W, b.reshape(1, D))
    ei_flat = edge_index.reshape(2 * E)
    partial, outbuf = _sc_aggregate(h, ei_flat, edge_weight, previous_index)
    return _combine(outbuf, partial)
